# baseline (speedup 1.0000x reference)
import functools

import jax
import jax.numpy as jnp
import numpy as np
from jax import lax
from jax.experimental import pallas as pl
from jax.experimental.pallas import tpu as pltpu
from jax.experimental.pallas import tpu_sc as plsc

NEMB = 128
NC = 2
NS = 16
NW = NC * NS
GCHUNK = 128


def _make_gather3(chunk, k):
    b_per_w = chunk // NW
    n_chunks = b_per_w // GCHUNK
    mesh = plsc.VectorSubcoreMesh(core_axis_name="c", subcore_axis_name="s")
    out_row = jax.ShapeDtypeStruct((chunk, NEMB // 2), jnp.int32)
    row0 = k * (chunk // GCHUNK)

    @functools.partial(
        pl.kernel,
        out_type=(out_row, out_row, out_row),
        mesh=mesh,
        scratch_types=[
            pltpu.VMEM((n_chunks, GCHUNK), jnp.int32),
            pltpu.VMEM((n_chunks, GCHUNK), jnp.int32),
            pltpu.VMEM((GCHUNK, NEMB), jnp.int32),
            pltpu.VMEM((GCHUNK, NEMB), jnp.int32),
            pltpu.VMEM((GCHUNK, NEMB // 2), jnp.int32),
            pltpu.VMEM((GCHUNK, NEMB // 2), jnp.int32),
            pltpu.SemaphoreType.DMA,
            pltpu.SemaphoreType.DMA,
            pltpu.SemaphoreType.DMA,
            pltpu.SemaphoreType.DMA,
        ],
    )
    def gather3(ev_hbm, en_hbm, xid_hbm, pid_hbm, nid_hbm,
                out_x, out_p, out_n,
                idx_a, idx_b, rows_a, rows_b, pk_a, pk_b,
                gsem_a, gsem_b, ssem_a, ssem_b):
        wid = lax.axis_index("s") * NC + lax.axis_index("c")
        base = wid * b_per_w
        idxb = (idx_a, idx_b)
        rows = (rows_a, rows_b)
        pk = (pk_a, pk_b)
        gsems = (gsem_a, gsem_b)
        ssems = (ssem_a, ssem_b)

        jobs = ((ev_hbm, xid_hbm, out_x),
                (en_hbm, pid_hbm, out_p),
                (en_hbm, nid_hbm, out_n))
        seq = [(j, c) for j in range(3) for c in range(n_chunks)]
        nseq = len(seq)

        def stage_idx(j):
            pltpu.sync_copy(
                jobs[j][1].at[pl.ds(row0 + wid * n_chunks, n_chunks)],
                idxb[j % 2])

        def start_gather(g):
            j, c = seq[g]
            cp = pltpu.make_async_copy(
                jobs[j][0].at[idxb[j % 2].at[c]], rows[g % 2], gsems[g % 2])
            cp.start()
            return cp

        def convert(g):
            src = rows[g % 2]
            dst = pk[g % 2]
            rnd = jnp.int32(0x8000)
            hmask = jnp.int32(-65536)

            def body(r, carry):
                for c4 in range(NEMB // 32):
                    a = src[r, pl.ds(c4 * 16, 16)]
                    bv = src[r, pl.ds(64 + c4 * 16, 16)]
                    lo = lax.shift_right_logical(a + rnd, 16)
                    hi = jnp.bitwise_and(bv + rnd, hmask)
                    dst[r, pl.ds(c4 * 16, 16)] = jnp.bitwise_or(lo, hi)
                return carry

            lax.fori_loop(0, GCHUNK, body, 0)

        def start_store(g):
            j, c = seq[g]
            cp = pltpu.make_async_copy(
                pk[g % 2],
                jobs[j][2].at[pl.ds(base + c * GCHUNK, GCHUNK)],
                ssems[g % 2])
            cp.start()
            return cp

        gh = [None] * nseq
        sh = [None] * nseq
        staged = set()

        def ensure_idx(j):
            if j not in staged:
                stage_idx(j)
                staged.add(j)

        ensure_idx(seq[0][0])
        gh[0] = start_gather(0)
        if nseq > 1:
            ensure_idx(seq[1][0])
            gh[1] = start_gather(1)
        for g in range(nseq):
            gh[g].wait()
            if g >= 2:
                sh[g - 2].wait()
            convert(g)
            sh[g] = start_store(g)
            nxt = g + 2
            if nxt < nseq:
                ensure_idx(seq[nxt][0])
                gh[nxt] = start_gather(nxt)
        if nseq >= 2:
            sh[nseq - 2].wait()
        sh[nseq - 1].wait()

    return gather3


def _make_bilinear_loss(batch):
    tb = 2048
    grid = (batch // tb,)
    half = NEMB // 2

    def unpack(w32):
        lo = lax.bitcast_convert_type(
            jnp.left_shift(w32, 16), jnp.float32)
        hi = lax.bitcast_convert_type(
            jnp.bitwise_and(w32, jnp.int32(-65536)), jnp.float32)
        return jnp.concatenate([lo, hi], axis=1)

    def body(x_ref, p_ref, n_ref, w_ref, b_ref, out_ref):
        i = pl.program_id(0)
        x = unpack(x_ref[...])
        p = unpack(p_ref[...])
        n = unpack(n_ref[...])
        u = jnp.dot(x, w_ref[...], preferred_element_type=jnp.float32)
        bias = b_ref[0]
        rr = lax.broadcasted_iota(jnp.int32, (NEMB, NEMB), 0)
        cc = lax.broadcasted_iota(jnp.int32, (NEMB, NEMB), 1)
        eye = (rr == cc).astype(jnp.float32)
        d1s, d2s = [], []
        for c in range(tb // NEMB):
            uc = lax.slice(u, (c * NEMB, 0), ((c + 1) * NEMB, NEMB))
            pc = lax.slice(p, (c * NEMB, 0), ((c + 1) * NEMB, NEMB))
            nc = lax.slice(n, (c * NEMB, 0), ((c + 1) * NEMB, NEMB))
            m1 = lax.dot_general(uc, pc, (((1,), (1,)), ((), ())),
                                 preferred_element_type=jnp.float32)
            m2 = lax.dot_general(uc, nc, (((1,), (1,)), ((), ())),
                                 preferred_element_type=jnp.float32)
            d1s.append(jnp.sum(m1 * eye, axis=0, keepdims=True))
            d2s.append(jnp.sum(m2 * eye, axis=0, keepdims=True))
        d1 = jnp.concatenate(d1s, axis=0) + bias
        d2 = jnp.concatenate(d2s, axis=0) + bias
        l1 = jnp.maximum(d1, 0.0) - d1 + jnp.log(1.0 + jnp.exp(-jnp.abs(d1)))
        l2 = jnp.maximum(d2, 0.0) + jnp.log(1.0 + jnp.exp(-jnp.abs(d2)))
        part = jnp.sum(l1 + l2)

        @pl.when(i == 0)
        def _():
            out_ref[0] = 0.0

        out_ref[0] += part

    return pl.pallas_call(
        body,
        grid=grid,
        in_specs=[
            pl.BlockSpec((tb, half), lambda i: (i, 0)),
            pl.BlockSpec((tb, half), lambda i: (i, 0)),
            pl.BlockSpec((tb, half), lambda i: (i, 0)),
            pl.BlockSpec((NEMB, NEMB), lambda i: (0, 0)),
            pl.BlockSpec(memory_space=pltpu.SMEM),
        ],
        out_specs=pl.BlockSpec(memory_space=pltpu.SMEM),
        out_shape=jax.ShapeDtypeStruct((1,), jnp.float32),
    )


def kernel(emb_event, emb_entity, W, b, x_id, pos_id, neg_id):
    batch = x_id.shape[0]
    nsplit = 2
    chunk = batch // nsplit
    tc_loss = _make_bilinear_loss(chunk)
    xi = x_id.astype(jnp.int32).reshape(batch // GCHUNK, GCHUNK)
    pi = pos_id.astype(jnp.int32).reshape(batch // GCHUNK, GCHUNK)
    ni = neg_id.astype(jnp.int32).reshape(batch // GCHUNK, GCHUNK)
    w0 = W[0]
    ev_i = lax.bitcast_convert_type(emb_event, jnp.int32)
    en_i = lax.bitcast_convert_type(emb_entity, jnp.int32)
    total = None
    for k in range(nsplit):
        x_g, p_g, n_g = _make_gather3(chunk, k)(
            ev_i, en_i, xi, pi, ni)
        part = tc_loss(x_g, p_g, n_g, w0, b)[0]
        total = part if total is None else total + part
    return total * (0.5 / batch)


# speedup vs baseline: 6.3111x; 6.3111x over previous
import functools

import jax
import jax.numpy as jnp
from jax import lax
from jax.experimental import pallas as pl
from jax.experimental.pallas import tpu as pltpu
from jax.experimental.pallas import tpu_sc as plsc

NEMB = 128
NC = 2
NS = 16
NW = NC * NS
GCHUNK = 128
NBUF = 3


def _make_gather3(chunk, k, nsplit):
    b_per_w = chunk // NW
    n_chunks = max(1, b_per_w // GCHUNK)
    gchunk = b_per_w // n_chunks
    mesh = plsc.VectorSubcoreMesh(core_axis_name="c", subcore_axis_name="s")
    out_row = jax.ShapeDtypeStruct((chunk, NEMB), jnp.float32)
    row0 = k * (chunk // gchunk)

    @functools.partial(
        pl.kernel,
        out_type=(out_row, out_row, out_row),
        mesh=mesh,
        scratch_types=[
            pltpu.VMEM((n_chunks, gchunk), jnp.int32),
            pltpu.VMEM((n_chunks, gchunk), jnp.int32),
            pltpu.VMEM((n_chunks, gchunk), jnp.int32),
            pltpu.VMEM((gchunk, NEMB), jnp.float32),
            pltpu.VMEM((gchunk, NEMB), jnp.float32),
            pltpu.VMEM((gchunk, NEMB), jnp.float32),
            pltpu.SemaphoreType.DMA,
            pltpu.SemaphoreType.DMA,
            pltpu.SemaphoreType.DMA,
            pltpu.SemaphoreType.DMA,
            pltpu.SemaphoreType.DMA,
            pltpu.SemaphoreType.DMA,
        ],
    )
    def gather3(ev_hbm, en_hbm, xid_hbm, pid_hbm, nid_hbm,
                out_x, out_p, out_n,
                idx_a, idx_b, idx_c, rows_a, rows_b, rows_c,
                gsem_a, gsem_b, gsem_c, ssem_a, ssem_b, ssem_c):
        wid = lax.axis_index("s") * NC + lax.axis_index("c")
        base = wid * b_per_w
        idxb = (idx_a, idx_b, idx_c)
        rows = (rows_a, rows_b, rows_c)
        gsems = (gsem_a, gsem_b, gsem_c)
        ssems = (ssem_a, ssem_b, ssem_c)

        jobs = ((ev_hbm, xid_hbm, out_x),
                (en_hbm, pid_hbm, out_p),
                (en_hbm, nid_hbm, out_n))
        seq = [(j, c) for j in range(3) for c in range(n_chunks)]
        nseq = len(seq)

        for j in range(3):
            pltpu.sync_copy(
                jobs[j][1].at[pl.ds(row0 + wid * n_chunks, n_chunks)],
                idxb[j])

        def start_gather(g):
            j, c = seq[g]
            cp = pltpu.make_async_copy(
                jobs[j][0].at[idxb[j].at[c]], rows[g % NBUF],
                gsems[g % NBUF])
            cp.start()
            return cp

        def start_store(g):
            j, c = seq[g]
            cp = pltpu.make_async_copy(
                rows[g % NBUF],
                jobs[j][2].at[pl.ds(base + c * gchunk, gchunk)],
                ssems[g % NBUF])
            cp.start()
            return cp

        gh = [None] * nseq
        sh = [None] * nseq
        for g in range(min(NBUF, nseq)):
            gh[g] = start_gather(g)
        for g in range(nseq):
            gh[g].wait()
            sh[g] = start_store(g)
            nxt = g + NBUF
            if nxt < nseq:
                sh[nxt - NBUF].wait()
                gh[nxt] = start_gather(nxt)
        for g in range(max(0, nseq - NBUF), nseq):
            if sh[g] is not None:
                sh[g].wait()

    return gather3


def _make_bilinear_loss(batch):
    tb = min(2048, batch)
    grid = (batch // tb,)

    def body(x_ref, p_ref, n_ref, w_ref, b_ref, out_ref):
        i = pl.program_id(0)
        u = jnp.dot(x_ref[...], w_ref[...],
                    preferred_element_type=jnp.float32)
        bias = b_ref[0]
        rr = lax.broadcasted_iota(jnp.int32, (NEMB, NEMB), 0)
        cc = lax.broadcasted_iota(jnp.int32, (NEMB, NEMB), 1)
        eye = (rr == cc).astype(jnp.float32)
        d1s, d2s = [], []
        for c in range(tb // NEMB):
            uc = lax.slice(u, (c * NEMB, 0), ((c + 1) * NEMB, NEMB))
            pc = p_ref[pl.ds(c * NEMB, NEMB), :]
            nc = n_ref[pl.ds(c * NEMB, NEMB), :]
            m1 = lax.dot_general(uc, pc, (((1,), (1,)), ((), ())),
                                 preferred_element_type=jnp.float32)
            m2 = lax.dot_general(uc, nc, (((1,), (1,)), ((), ())),
                                 preferred_element_type=jnp.float32)
            d1s.append(jnp.sum(m1 * eye, axis=0, keepdims=True))
            d2s.append(jnp.sum(m2 * eye, axis=0, keepdims=True))
        d1 = jnp.concatenate(d1s, axis=0) + bias
        d2 = jnp.concatenate(d2s, axis=0) + bias
        l1 = jnp.maximum(d1, 0.0) - d1 + jnp.log(1.0 + jnp.exp(-jnp.abs(d1)))
        l2 = jnp.maximum(d2, 0.0) + jnp.log(1.0 + jnp.exp(-jnp.abs(d2)))
        part = jnp.sum(l1 + l2)

        @pl.when(i == 0)
        def _():
            out_ref[0] = 0.0

        out_ref[0] += part

    return pl.pallas_call(
        body,
        grid=grid,
        in_specs=[
            pl.BlockSpec((tb, NEMB), lambda i: (i, 0)),
            pl.BlockSpec((tb, NEMB), lambda i: (i, 0)),
            pl.BlockSpec((tb, NEMB), lambda i: (i, 0)),
            pl.BlockSpec((NEMB, NEMB), lambda i: (0, 0)),
            pl.BlockSpec(memory_space=pltpu.SMEM),
        ],
        out_specs=pl.BlockSpec(memory_space=pltpu.SMEM),
        out_shape=jax.ShapeDtypeStruct((1,), jnp.float32),
    )


def kernel(emb_event, emb_entity, W, b, x_id, pos_id, neg_id):
    batch = x_id.shape[0]
    nsplit = 4
    chunk = batch // nsplit
    tc_loss = _make_bilinear_loss(chunk)
    gchunk = min(GCHUNK, chunk // NW)
    xi = x_id.astype(jnp.int32).reshape(batch // gchunk, gchunk)
    pi = pos_id.astype(jnp.int32).reshape(batch // gchunk, gchunk)
    ni = neg_id.astype(jnp.int32).reshape(batch // gchunk, gchunk)
    w0 = W[0]
    total = None
    for k in range(nsplit):
        x_g, p_g, n_g = _make_gather3(chunk, k, nsplit)(
            emb_event, emb_entity, xi, pi, ni)
        part = tc_loss(x_g, p_g, n_g, w0, b)[0]
        total = part if total is None else total + part
    return total * (0.5 / batch)


# speedup vs baseline: 7.0730x; 1.1207x over previous
import functools

import jax
import jax.numpy as jnp
from jax import lax
from jax.experimental import pallas as pl
from jax.experimental.pallas import tpu as pltpu
from jax.experimental.pallas import tpu_sc as plsc

NEMB = 128
NC = 2
NS = 16
NW = NC * NS
GCHUNK = 128
NBUF = 3


def _make_gather3(chunk, k, nsplit):
    b_per_w = chunk // NW
    n_chunks = max(1, b_per_w // GCHUNK)
    gchunk = b_per_w // n_chunks
    mesh = plsc.VectorSubcoreMesh(core_axis_name="c", subcore_axis_name="s")
    out_row = jax.ShapeDtypeStruct((chunk, NEMB), jnp.float32)
    row0 = k * (chunk // gchunk)

    @functools.partial(
        pl.kernel,
        out_type=(out_row, out_row, out_row),
        mesh=mesh,
        scratch_types=[
            pltpu.VMEM((n_chunks, gchunk), jnp.int32),
            pltpu.VMEM((n_chunks, gchunk), jnp.int32),
            pltpu.VMEM((n_chunks, gchunk), jnp.int32),
            pltpu.VMEM((gchunk, NEMB), jnp.float32),
            pltpu.VMEM((gchunk, NEMB), jnp.float32),
            pltpu.VMEM((gchunk, NEMB), jnp.float32),
            pltpu.SemaphoreType.DMA,
            pltpu.SemaphoreType.DMA,
            pltpu.SemaphoreType.DMA,
            pltpu.SemaphoreType.DMA,
            pltpu.SemaphoreType.DMA,
            pltpu.SemaphoreType.DMA,
        ],
    )
    def gather3(ev_hbm, en_hbm, xid_hbm, pid_hbm, nid_hbm,
                out_x, out_p, out_n,
                idx_a, idx_b, idx_c, rows_a, rows_b, rows_c,
                gsem_a, gsem_b, gsem_c, ssem_a, ssem_b, ssem_c):
        wid = lax.axis_index("s") * NC + lax.axis_index("c")
        base = wid * b_per_w
        idxb = (idx_a, idx_b, idx_c)
        rows = (rows_a, rows_b, rows_c)
        gsems = (gsem_a, gsem_b, gsem_c)
        ssems = (ssem_a, ssem_b, ssem_c)

        jobs = ((ev_hbm, xid_hbm, out_x),
                (en_hbm, pid_hbm, out_p),
                (en_hbm, nid_hbm, out_n))
        seq = [(j, c) for j in range(3) for c in range(n_chunks)]
        nseq = len(seq)

        for j in range(3):
            pltpu.sync_copy(
                jobs[j][1].at[pl.ds(row0 + wid * n_chunks, n_chunks)],
                idxb[j])

        def start_gather(g):
            j, c = seq[g]
            cp = pltpu.make_async_copy(
                jobs[j][0].at[idxb[j].at[c]], rows[g % NBUF],
                gsems[g % NBUF])
            cp.start()
            return cp

        def start_store(g):
            j, c = seq[g]
            cp = pltpu.make_async_copy(
                rows[g % NBUF],
                jobs[j][2].at[pl.ds(base + c * gchunk, gchunk)],
                ssems[g % NBUF])
            cp.start()
            return cp

        gh = [None] * nseq
        sh = [None] * nseq
        for g in range(min(NBUF, nseq)):
            gh[g] = start_gather(g)
        for g in range(nseq):
            gh[g].wait()
            sh[g] = start_store(g)
            nxt = g + NBUF
            if nxt < nseq:
                sh[nxt - NBUF].wait()
                gh[nxt] = start_gather(nxt)
        for g in range(max(0, nseq - NBUF), nseq):
            if sh[g] is not None:
                sh[g].wait()

    return gather3


def _make_bilinear_loss(batch):
    tb = min(2048, batch)
    grid = (batch // tb,)

    def body(x_ref, p_ref, n_ref, w_ref, b_ref, out_ref):
        i = pl.program_id(0)
        u = jnp.dot(x_ref[...], w_ref[...],
                    preferred_element_type=jnp.float32)
        bias = b_ref[0]
        rr = lax.broadcasted_iota(jnp.int32, (NEMB, NEMB), 0)
        cc = lax.broadcasted_iota(jnp.int32, (NEMB, NEMB), 1)
        eye = (rr == cc).astype(jnp.float32)
        d1s, d2s = [], []
        for c in range(tb // NEMB):
            uc = lax.slice(u, (c * NEMB, 0), ((c + 1) * NEMB, NEMB))
            pc = p_ref[pl.ds(c * NEMB, NEMB), :]
            nc = n_ref[pl.ds(c * NEMB, NEMB), :]
            m1 = lax.dot_general(uc, pc, (((1,), (1,)), ((), ())),
                                 preferred_element_type=jnp.float32)
            m2 = lax.dot_general(uc, nc, (((1,), (1,)), ((), ())),
                                 preferred_element_type=jnp.float32)
            d1s.append(jnp.sum(m1 * eye, axis=0, keepdims=True))
            d2s.append(jnp.sum(m2 * eye, axis=0, keepdims=True))
        d1 = jnp.concatenate(d1s, axis=0) + bias
        d2 = jnp.concatenate(d2s, axis=0) + bias
        l1 = jnp.maximum(d1, 0.0) - d1 + jnp.log(1.0 + jnp.exp(-jnp.abs(d1)))
        l2 = jnp.maximum(d2, 0.0) + jnp.log(1.0 + jnp.exp(-jnp.abs(d2)))
        part = jnp.sum(l1 + l2)

        @pl.when(i == 0)
        def _():
            out_ref[0] = 0.0

        out_ref[0] += part

    return pl.pallas_call(
        body,
        grid=grid,
        in_specs=[
            pl.BlockSpec((tb, NEMB), lambda i: (i, 0)),
            pl.BlockSpec((tb, NEMB), lambda i: (i, 0)),
            pl.BlockSpec((tb, NEMB), lambda i: (i, 0)),
            pl.BlockSpec((NEMB, NEMB), lambda i: (0, 0)),
            pl.BlockSpec(memory_space=pltpu.SMEM),
        ],
        out_specs=pl.BlockSpec(memory_space=pltpu.SMEM),
        out_shape=jax.ShapeDtypeStruct((1,), jnp.float32),
    )


def kernel(emb_event, emb_entity, W, b, x_id, pos_id, neg_id):
    batch = x_id.shape[0]
    nsplit = 2
    chunk = batch // nsplit
    tc_loss = _make_bilinear_loss(chunk)
    gchunk = min(GCHUNK, chunk // NW)
    xi = x_id.astype(jnp.int32).reshape(batch // gchunk, gchunk)
    pi = pos_id.astype(jnp.int32).reshape(batch // gchunk, gchunk)
    ni = neg_id.astype(jnp.int32).reshape(batch // gchunk, gchunk)
    w0 = W[0]
    total = None
    for k in range(nsplit):
        x_g, p_g, n_g = _make_gather3(chunk, k, nsplit)(
            emb_event, emb_entity, xi, pi, ni)
        part = tc_loss(x_g, p_g, n_g, w0, b)[0]
        total = part if total is None else total + part
    return total * (0.5 / batch)
